# baseline (speedup 1.0000x reference)
import functools

import jax
import jax.numpy as jnp
from jax import lax
from jax.experimental import pallas as pl
from jax.experimental.pallas import tpu as pltpu
from jax.experimental.pallas import tpu_sc as plsc


def _make_sc_gather(n_rows, emb):
  info = plsc.get_sparse_core_info()
  nc, ns = info.num_cores, info.num_subcores
  nw = nc * ns
  b_per_w = n_rows // nw
  ch = 128
  n_ch = b_per_w // ch
  fire = max(f for f in range(1, min(16, n_ch) + 1) if n_ch % f == 0)
  mesh = plsc.VectorSubcoreMesh(core_axis_name="c", subcore_axis_name="s")

  @functools.partial(
      pl.kernel, mesh=mesh,
      out_type=jax.ShapeDtypeStruct((n_rows, emb), jnp.float32),
      compiler_params=pltpu.CompilerParams(use_tc_tiling_on_sc=False),
      scratch_types=[
          pltpu.VMEM((n_ch, ch), jnp.int32),
          pltpu.VMEM((fire * ch, emb), jnp.float32),
          pltpu.SemaphoreType.DMA,
      ],
  )
  def gather(table_hbm, idx_hbm, out_hbm, idx_v, rows_v, sem):
    wid = lax.axis_index("s") * nc + lax.axis_index("c")
    base = wid * b_per_w
    pltpu.sync_copy(idx_hbm.at[wid], idx_v)

    def round_(r):
      for k in range(fire):
        pltpu.async_copy(table_hbm.at[idx_v.at[r * fire + k]],
                         rows_v.at[pl.ds(k * ch, ch)], sem)
      for k in range(fire):
        pltpu.make_async_copy(table_hbm.at[idx_v.at[0]],
                              rows_v.at[pl.ds(k * ch, ch)], sem).wait()
      pltpu.sync_copy(rows_v,
                      out_hbm.at[pl.ds(base + r * fire * ch, fire * ch)])

    pl.loop(0, n_ch // fire)(round_)

  return gather, nw, n_ch, ch


def _lm_head(x3, pos, wt, bcol, b_blk, n_b_total, blk_off, prev=None):
  n_b, t_dim, emb = x3.shape
  vocab = wt.shape[0]
  grid = n_b // b_blk

  def body(*refs):
    x_ref, p_ref, w_ref, b_ref = refs[-5:-1]
    o_ref = refs[-1]
    for t in range(t_dim):
      xt = x_ref[:, t, :] + p_ref[t, :][None, :]
      o_ref[t] = (
          jax.lax.dot_general(
              w_ref[...], xt, (((1,), (1,)), ((), ())),
              preferred_element_type=jnp.float32)
          + b_ref[...]
      )

  in_specs = [
      pl.BlockSpec((b_blk, t_dim, emb), lambda i: (i, 0, 0)),
      pl.BlockSpec((t_dim, emb), lambda i: (0, 0)),
      pl.BlockSpec((vocab, emb), lambda i: (0, 0)),
      pl.BlockSpec((vocab, 1), lambda i: (0, 0)),
  ]
  args = [x3, pos, wt, bcol]
  aliases = {}
  if prev is not None:
    in_specs = [pl.BlockSpec(memory_space=pl.ANY)] + in_specs
    args = [prev] + args
    aliases = {0: 0}

  return pl.pallas_call(
      body,
      grid=(grid,),
      in_specs=in_specs,
      out_specs=pl.BlockSpec((t_dim, vocab, b_blk),
                             lambda i: (0, 0, i + blk_off)),
      out_shape=jax.ShapeDtypeStruct((t_dim, vocab, n_b_total), jnp.float32),
      input_output_aliases=aliases,
      compiler_params=pltpu.CompilerParams(
          dimension_semantics=("arbitrary",),
      ),
  )(*args)


def kernel(idx, tok_table, pos_table, W, b):
  B, T = idx.shape
  vocab, emb = tok_table.shape
  b_blk = 512
  chunks = (4096, B - 4096)

  wt, bcol = W.T, b.reshape(vocab, 1)
  out = None
  off = 0
  for nb in chunks:
    sc_gather, nw, n_ch, ch = _make_sc_gather(nb * T, emb)
    idx_c = idx[off:off + nb].reshape(nw, n_ch, ch)
    x_c = sc_gather(tok_table, idx_c).reshape(nb, T, emb)
    out = _lm_head(x_c, pos_table, wt, bcol, b_blk, B, off // b_blk, prev=out)
    off += nb
  return out.transpose(2, 0, 1)


# speedup vs baseline: 1.0288x; 1.0288x over previous
import functools

import jax
import jax.numpy as jnp
from jax import lax
from jax.experimental import pallas as pl
from jax.experimental.pallas import tpu as pltpu
from jax.experimental.pallas import tpu_sc as plsc


def _make_sc_gather(n_rows, emb):
  info = plsc.get_sparse_core_info()
  nc, ns = info.num_cores, info.num_subcores
  nw = nc * ns
  b_per_w = n_rows // nw
  ch = 128
  n_ch = b_per_w // ch
  fire = max(f for f in range(1, min(16, n_ch) + 1) if n_ch % f == 0)
  mesh = plsc.VectorSubcoreMesh(core_axis_name="c", subcore_axis_name="s")

  @functools.partial(
      pl.kernel, mesh=mesh,
      out_type=jax.ShapeDtypeStruct((n_rows, emb), jnp.float32),
      compiler_params=pltpu.CompilerParams(use_tc_tiling_on_sc=False),
      scratch_types=[
          pltpu.VMEM((n_ch, ch), jnp.int32),
          pltpu.VMEM((fire * ch, emb), jnp.float32),
          pltpu.SemaphoreType.DMA,
      ],
  )
  def gather(table_hbm, idx_hbm, out_hbm, idx_v, rows_v, sem):
    wid = lax.axis_index("s") * nc + lax.axis_index("c")
    base = wid * b_per_w
    pltpu.sync_copy(idx_hbm.at[wid], idx_v)

    def round_(r):
      for k in range(fire):
        pltpu.async_copy(table_hbm.at[idx_v.at[r * fire + k]],
                         rows_v.at[pl.ds(k * ch, ch)], sem)
      for k in range(fire):
        pltpu.make_async_copy(table_hbm.at[idx_v.at[0]],
                              rows_v.at[pl.ds(k * ch, ch)], sem).wait()
      pltpu.sync_copy(rows_v,
                      out_hbm.at[pl.ds(base + r * fire * ch, fire * ch)])

    pl.loop(0, n_ch // fire)(round_)

  return gather, nw, n_ch, ch


def _lm_head(x3, pos, wt, bcol, b_blk, n_b_total, blk_off, prev=None):
  n_b, t_dim, emb = x3.shape
  vocab = wt.shape[0]
  t_blk = t_dim
  grid = (1, n_b // b_blk)

  def body(*refs):
    x_ref, p_ref, w_ref, b_ref = refs[-5:-1]
    o_ref = refs[-1]
    for t in range(t_blk):
      xt = x_ref[:, t, :] + p_ref[t, :][None, :]
      o_ref[t] = (
          jax.lax.dot_general(
              w_ref[...], xt, (((1,), (1,)), ((), ())),
              preferred_element_type=jnp.float32)
          + b_ref[...]
      )

  in_specs = [
      pl.BlockSpec((b_blk, t_blk, emb), lambda j, i: (i, j, 0)),
      pl.BlockSpec((t_blk, emb), lambda j, i: (j, 0)),
      pl.BlockSpec((vocab, emb), lambda j, i: (0, 0)),
      pl.BlockSpec((vocab, 1), lambda j, i: (0, 0)),
  ]
  args = [x3, pos, wt, bcol]
  aliases = {}
  if prev is not None:
    in_specs = [pl.BlockSpec(memory_space=pl.ANY)] + in_specs
    args = [prev] + args
    aliases = {0: 0}

  return pl.pallas_call(
      body,
      grid=grid,
      in_specs=in_specs,
      out_specs=pl.BlockSpec((t_blk, vocab, b_blk),
                             lambda j, i: (j, 0, i + blk_off)),
      out_shape=jax.ShapeDtypeStruct((t_dim, vocab, n_b_total), jnp.float32),
      input_output_aliases=aliases,
      compiler_params=pltpu.CompilerParams(
          dimension_semantics=("arbitrary", "arbitrary"),
      ),
  )(*args)


def kernel(idx, tok_table, pos_table, W, b):
  B, T = idx.shape
  vocab, emb = tok_table.shape
  b_blk = 512
  chunks = (B // 2, B // 2)

  wt, bcol = W.T, b.reshape(vocab, 1)
  out = None
  off = 0
  for nb in chunks:
    sc_gather, nw, n_ch, ch = _make_sc_gather(nb * T, emb)
    idx_c = idx[off:off + nb].reshape(nw, n_ch, ch)
    x_c = sc_gather(tok_table, idx_c).reshape(nb, T, emb)
    out = _lm_head(x_c, pos_table, wt, bcol, b_blk, B, off // b_blk, prev=out)
    off += nb
  return out.transpose(2, 0, 1)
